# baseline (speedup 1.0000x reference)
import functools

import jax
import jax.numpy as jnp
from jax import lax
from jax.experimental import pallas as pl
from jax.experimental.pallas import tpu as pltpu
from jax.experimental.pallas import tpu_sc as plsc

_N_TOKENS = 16384
_NUM_EMB = 8192
_DIM = 32
_TB = 512


_LANES = 128
_TCHUNK = _NUM_EMB * _TB // _N_TOKENS


def _argmin_body_notable(x_ref, e_ref, idx_ref, em2_ref, e2_ref):
    _argmin_core(x_ref, e_ref, idx_ref, em2_ref, e2_ref)


def _argmin_body(x_ref, e_ref, tbl_ref, idx_ref, em2_ref, e2_ref):
    i = pl.program_id(0)
    tchunk = tbl_ref.shape[0]
    tbl_ref[...] = e_ref[pl.ds(0, _DIM), pl.ds(i * tchunk, tchunk)].T
    _argmin_core(x_ref, e_ref, idx_ref, em2_ref, e2_ref)


def _argmin_core(x_ref, e_ref, idx_ref, em2_ref, e2_ref):
    @pl.when(pl.program_id(0) == 0)
    def _init():
        e = e_ref[...]
        em2_ref[...] = e * -2.0
        e2_ref[...] = jnp.broadcast_to(
            jnp.sum(e * e, axis=0, keepdims=True), (8, _NUM_EMB)
        )

    xd = jnp.dot(
        x_ref[...], em2_ref[...], preferred_element_type=jnp.float32
    )
    d = xd + e2_ref[0:1, :]

    run_min = d[:, :_LANES]
    run_cid = jnp.zeros((_TB, _LANES), jnp.float32)
    for c in range(1, _NUM_EMB // _LANES):
        dc = d[:, c * _LANES:(c + 1) * _LANES]
        pred = dc < run_min
        run_min = jnp.where(pred, dc, run_min)
        run_cid = jnp.where(pred, jnp.float32(c), run_cid)

    m = jnp.min(run_min, axis=1, keepdims=True)
    lane = lax.broadcasted_iota(
        jnp.int32, (_TB, _LANES), 1
    ).astype(jnp.float32)
    cand = jnp.where(
        run_min == m, run_cid * _LANES + lane, jnp.float32(_NUM_EMB)
    )
    idx_ref[0, 0, :] = jnp.min(cand, axis=1).astype(jnp.int32)


def _tc_argmin(x, embeddings, emit_table):
    n = x.shape[0]
    nb = n // _TB
    body = _argmin_body if emit_table else _argmin_body_notable
    tchunk = _NUM_EMB // nb
    out_specs = [pl.BlockSpec((1, 1, _TB), lambda i: (i, 0, 0))]
    out_shape = [jax.ShapeDtypeStruct((nb, 1, _TB), jnp.int32)]
    if emit_table:
        out_specs.insert(0, pl.BlockSpec((tchunk, _DIM), lambda i: (i, 0)))
        out_shape.insert(0, jax.ShapeDtypeStruct((_NUM_EMB, _DIM), jnp.float32))
    out = pl.pallas_call(
        body,
        grid=(nb,),
        in_specs=[
            pl.BlockSpec((_TB, _DIM), lambda i: (i, 0)),
            pl.BlockSpec((_DIM, _NUM_EMB), lambda i: (0, 0)),
        ],
        out_specs=out_specs,
        out_shape=out_shape,
        scratch_shapes=[
            pltpu.VMEM((_DIM, _NUM_EMB), jnp.float32),
            pltpu.VMEM((8, _NUM_EMB), jnp.float32),
        ],
    )(x, embeddings)
    if emit_table:
        return out[0], out[1].reshape(n)
    return out[0].reshape(n)


def _sc_gather(table, idx):
    info = plsc.get_sparse_core_info()
    nc, ns = info.num_cores, info.num_subcores
    nw = nc * ns
    n = idx.shape[0]
    bpw = n // nw
    mesh = plsc.VectorSubcoreMesh(core_axis_name="c", subcore_axis_name="s")

    @functools.partial(
        pl.kernel,
        mesh=mesh,
        compiler_params=pltpu.CompilerParams(use_tc_tiling_on_sc=False),
        out_type=jax.ShapeDtypeStruct((n, _DIM), jnp.float32),
        scratch_types=[
            pltpu.VMEM((bpw,), jnp.int32),
            pltpu.VMEM((bpw, _DIM), jnp.float32),
            pltpu.SemaphoreType.DMA,
        ],
    )
    def gather_kernel(table_hbm, idx_hbm, out_hbm, idx_v, rows_v, sem):
        wid = lax.axis_index("s") * nc + lax.axis_index("c")
        base = wid * bpw
        pltpu.sync_copy(idx_hbm.at[pl.ds(base, bpw)], idx_v)
        pltpu.async_copy(table_hbm.at[idx_v], rows_v, sem).wait()
        pltpu.sync_copy(rows_v, out_hbm.at[pl.ds(base, bpw)])

    return gather_kernel(table, idx)


def kernel(x, embeddings):
    h = _N_TOKENS // 2
    table, idx_a = _tc_argmin(x[:h], embeddings, emit_table=True)
    idx_b = _tc_argmin(x[h:], embeddings, emit_table=False)
    qa = _sc_gather(table, idx_a)
    qb = _sc_gather(table, idx_b)
    return jnp.concatenate([qa, qb], axis=0)


# speedup vs baseline: 1.1175x; 1.1175x over previous
import functools

import jax
import jax.numpy as jnp
from jax import lax
from jax.experimental import pallas as pl
from jax.experimental.pallas import tpu as pltpu
from jax.experimental.pallas import tpu_sc as plsc

_N_TOKENS = 16384
_NUM_EMB = 8192
_DIM = 32
_TB = 512


_LANES = 128
_TCHUNK = _NUM_EMB * _TB // _N_TOKENS


def _argmin_body(x_ref, e_ref, tbl_ref, idx_ref, em2_ref, e2_ref):
    @pl.when(pl.program_id(0) == 0)
    def _init():
        e = e_ref[...]
        em2_ref[...] = e * -2.0
        e2_ref[...] = jnp.broadcast_to(
            jnp.sum(e * e, axis=0, keepdims=True), (8, _NUM_EMB)
        )

    i = pl.program_id(0)
    tbl_ref[...] = e_ref[pl.ds(0, _DIM), pl.ds(i * _TCHUNK, _TCHUNK)].T

    xd = jnp.dot(
        x_ref[...], em2_ref[...], preferred_element_type=jnp.float32
    )
    d = xd + e2_ref[0:1, :]

    run_min = d[:, :_LANES]
    run_cid = jnp.zeros((_TB, _LANES), jnp.float32)
    for c in range(1, _NUM_EMB // _LANES):
        dc = d[:, c * _LANES:(c + 1) * _LANES]
        pred = dc < run_min
        run_min = jnp.where(pred, dc, run_min)
        run_cid = jnp.where(pred, jnp.float32(c), run_cid)

    m = jnp.min(run_min, axis=1, keepdims=True)
    lane = lax.broadcasted_iota(
        jnp.int32, (_TB, _LANES), 1
    ).astype(jnp.float32)
    cand = jnp.where(
        run_min == m, run_cid * _LANES + lane, jnp.float32(_NUM_EMB)
    )
    idxf = jnp.min(cand, axis=1)
    idx_ref[0] = idxf.astype(jnp.int32).reshape(_TB // _LANES, _LANES)


def _tc_argmin(x, embeddings):
    nb = _N_TOKENS // _TB
    out = pl.pallas_call(
        _argmin_body,
        grid=(nb,),
        in_specs=[
            pl.BlockSpec((_TB, _DIM), lambda i: (i, 0)),
            pl.BlockSpec((_DIM, _NUM_EMB), lambda i: (0, 0)),
        ],
        out_specs=[
            pl.BlockSpec((_TCHUNK, _DIM), lambda i: (i, 0)),
            pl.BlockSpec((1, _TB // _LANES, _LANES), lambda i: (i, 0, 0)),
        ],
        out_shape=[
            jax.ShapeDtypeStruct((_NUM_EMB, _DIM), jnp.float32),
            jax.ShapeDtypeStruct((nb, _TB // _LANES, _LANES), jnp.int32),
        ],
        scratch_shapes=[
            pltpu.VMEM((_DIM, _NUM_EMB), jnp.float32),
            pltpu.VMEM((8, _NUM_EMB), jnp.float32),
        ],
    )(x, embeddings)
    return out[0], out[1].reshape(_N_TOKENS)


def _sc_gather(table, idx):
    info = plsc.get_sparse_core_info()
    nc, ns = info.num_cores, info.num_subcores
    nw = nc * ns
    bpw = _N_TOKENS // nw
    mesh = plsc.VectorSubcoreMesh(core_axis_name="c", subcore_axis_name="s")

    @functools.partial(
        pl.kernel,
        mesh=mesh,
        compiler_params=pltpu.CompilerParams(use_tc_tiling_on_sc=False),
        out_type=jax.ShapeDtypeStruct((_N_TOKENS, _DIM), jnp.float32),
        scratch_types=[
            pltpu.VMEM((bpw,), jnp.int32),
            pltpu.VMEM((bpw, _DIM), jnp.float32),
            pltpu.SemaphoreType.DMA,
        ],
    )
    def gather_kernel(table_hbm, idx_hbm, out_hbm, idx_v, rows_v, sem):
        wid = lax.axis_index("s") * nc + lax.axis_index("c")
        base = wid * bpw
        pltpu.sync_copy(idx_hbm.at[pl.ds(base, bpw)], idx_v)
        pltpu.async_copy(table_hbm.at[idx_v], rows_v, sem).wait()
        pltpu.sync_copy(rows_v, out_hbm.at[pl.ds(base, bpw)])

    return gather_kernel(table, idx)


def kernel(x, embeddings):
    table, idx = _tc_argmin(x, embeddings)
    return _sc_gather(table, idx)


# speedup vs baseline: 1.1599x; 1.0379x over previous
import functools

import jax
import jax.numpy as jnp
from jax import lax
from jax.experimental import pallas as pl
from jax.experimental.pallas import tpu as pltpu
from jax.experimental.pallas import tpu_sc as plsc

_N_TOKENS = 16384
_NUM_EMB = 8192
_DIM = 32
_TB = 1024


_LANES = 128
_TCHUNK = _NUM_EMB * _TB // _N_TOKENS


def _argmin_body(x_ref, e_ref, tbl_ref, idx_ref, em2_ref, e2_ref):
    @pl.when(pl.program_id(0) == 0)
    def _init():
        e = e_ref[...]
        em2_ref[...] = e * -2.0
        e2_ref[...] = jnp.broadcast_to(
            jnp.sum(e * e, axis=0, keepdims=True), (8, _NUM_EMB)
        )

    i = pl.program_id(0)
    tbl_ref[...] = e_ref[pl.ds(0, _DIM), pl.ds(i * _TCHUNK, _TCHUNK)].T

    xd = jnp.dot(
        x_ref[...], em2_ref[...], preferred_element_type=jnp.float32
    )
    d = xd + e2_ref[0:1, :]

    run_min = d[:, :_LANES]
    run_cid = jnp.zeros((_TB, _LANES), jnp.float32)
    for c in range(1, _NUM_EMB // _LANES):
        dc = d[:, c * _LANES:(c + 1) * _LANES]
        pred = dc < run_min
        run_min = jnp.where(pred, dc, run_min)
        run_cid = jnp.where(pred, jnp.float32(c), run_cid)

    m = jnp.min(run_min, axis=1, keepdims=True)
    lane = lax.broadcasted_iota(
        jnp.int32, (_TB, _LANES), 1
    ).astype(jnp.float32)
    cand = jnp.where(
        run_min == m, run_cid * _LANES + lane, jnp.float32(_NUM_EMB)
    )
    idxf = jnp.min(cand, axis=1)
    idx_ref[0] = idxf.astype(jnp.int32).reshape(_TB // _LANES, _LANES)


def _tc_argmin(x, embeddings):
    nb = _N_TOKENS // _TB
    out = pl.pallas_call(
        _argmin_body,
        grid=(nb,),
        in_specs=[
            pl.BlockSpec((_TB, _DIM), lambda i: (i, 0)),
            pl.BlockSpec((_DIM, _NUM_EMB), lambda i: (0, 0)),
        ],
        out_specs=[
            pl.BlockSpec((_TCHUNK, _DIM), lambda i: (i, 0)),
            pl.BlockSpec((1, _TB // _LANES, _LANES), lambda i: (i, 0, 0)),
        ],
        out_shape=[
            jax.ShapeDtypeStruct((_NUM_EMB, _DIM), jnp.float32),
            jax.ShapeDtypeStruct((nb, _TB // _LANES, _LANES), jnp.int32),
        ],
        scratch_shapes=[
            pltpu.VMEM((_DIM, _NUM_EMB), jnp.float32),
            pltpu.VMEM((8, _NUM_EMB), jnp.float32),
        ],
    )(x, embeddings)
    return out[0], out[1].reshape(_N_TOKENS)


def _sc_gather(table, idx):
    info = plsc.get_sparse_core_info()
    nc, ns = info.num_cores, info.num_subcores
    nw = nc * ns
    bpw = _N_TOKENS // nw
    mesh = plsc.VectorSubcoreMesh(core_axis_name="c", subcore_axis_name="s")

    @functools.partial(
        pl.kernel,
        mesh=mesh,
        compiler_params=pltpu.CompilerParams(use_tc_tiling_on_sc=False),
        out_type=jax.ShapeDtypeStruct((_N_TOKENS, _DIM), jnp.float32),
        scratch_types=[
            pltpu.VMEM((bpw,), jnp.int32),
            pltpu.VMEM((bpw, _DIM), jnp.float32),
            pltpu.SemaphoreType.DMA,
        ],
    )
    def gather_kernel(table_hbm, idx_hbm, out_hbm, idx_v, rows_v, sem):
        wid = lax.axis_index("s") * nc + lax.axis_index("c")
        base = wid * bpw
        pltpu.sync_copy(idx_hbm.at[pl.ds(base, bpw)], idx_v)
        pltpu.async_copy(table_hbm.at[idx_v], rows_v, sem).wait()
        pltpu.sync_copy(rows_v, out_hbm.at[pl.ds(base, bpw)])

    return gather_kernel(table, idx)


def kernel(x, embeddings):
    table, idx = _tc_argmin(x, embeddings)
    return _sc_gather(table, idx)


# speedup vs baseline: 1.2215x; 1.0531x over previous
import functools

import jax
import jax.numpy as jnp
from jax import lax
from jax.experimental import pallas as pl
from jax.experimental.pallas import tpu as pltpu
from jax.experimental.pallas import tpu_sc as plsc

_N_TOKENS = 16384
_NUM_EMB = 8192
_DIM = 32
_TB = 1024


_LANES = 128
_TCHUNK = _NUM_EMB * _TB // _N_TOKENS


def _argmin_body(x_ref, e_ref, tbl_ref, idx_ref, em2_ref):
    @pl.when(pl.program_id(0) == 0)
    def _init():
        e = e_ref[...]
        em2_ref[0:_DIM] = e * -2.0
        e2 = jnp.sum(e * e, axis=0, keepdims=True)
        hi = e2.astype(jnp.bfloat16).astype(jnp.float32)
        r = e2 - hi
        mid = r.astype(jnp.bfloat16).astype(jnp.float32)
        lo = r - mid
        em2_ref[_DIM:_DIM + 1] = hi
        em2_ref[_DIM + 1:_DIM + 2] = mid
        em2_ref[_DIM + 2:_DIM + 3] = lo
        em2_ref[_DIM + 3:] = jnp.zeros((5, _NUM_EMB), jnp.float32)

    i = pl.program_id(0)
    tbl_ref[...] = e_ref[pl.ds(0, _DIM), pl.ds(i * _TCHUNK, _TCHUNK)].T

    ones3 = jnp.where(
        lax.broadcasted_iota(jnp.int32, (_TB, 8), 1) < 3,
        jnp.float32(1.0), jnp.float32(0.0),
    )
    xa = jnp.concatenate([x_ref[...], ones3], axis=1)
    d = jnp.dot(
        xa, em2_ref[...], preferred_element_type=jnp.float32
    )

    run_min = d[:, :_LANES]
    run_cid = jnp.zeros((_TB, _LANES), jnp.float32)
    for c in range(1, _NUM_EMB // _LANES):
        dc = d[:, c * _LANES:(c + 1) * _LANES]
        pred = dc < run_min
        run_min = jnp.where(pred, dc, run_min)
        run_cid = jnp.where(pred, jnp.float32(c), run_cid)

    m = jnp.min(run_min, axis=1, keepdims=True)
    lane = lax.broadcasted_iota(
        jnp.int32, (_TB, _LANES), 1
    ).astype(jnp.float32)
    cand = jnp.where(
        run_min == m, run_cid * _LANES + lane, jnp.float32(_NUM_EMB)
    )
    idxf = jnp.min(cand, axis=1)
    idx_ref[0] = idxf.astype(jnp.int32).reshape(_TB // _LANES, _LANES)


def _tc_argmin(x, embeddings):
    nb = _N_TOKENS // _TB
    out = pl.pallas_call(
        _argmin_body,
        grid=(nb,),
        in_specs=[
            pl.BlockSpec((_TB, _DIM), lambda i: (i, 0)),
            pl.BlockSpec((_DIM, _NUM_EMB), lambda i: (0, 0)),
        ],
        out_specs=[
            pl.BlockSpec((_TCHUNK, _DIM), lambda i: (i, 0)),
            pl.BlockSpec((1, _TB // _LANES, _LANES), lambda i: (i, 0, 0)),
        ],
        out_shape=[
            jax.ShapeDtypeStruct((_NUM_EMB, _DIM), jnp.float32),
            jax.ShapeDtypeStruct((nb, _TB // _LANES, _LANES), jnp.int32),
        ],
        scratch_shapes=[
            pltpu.VMEM((_DIM + 8, _NUM_EMB), jnp.float32),
        ],
    )(x, embeddings)
    return out[0], out[1].reshape(_N_TOKENS)


def _sc_gather(table, idx):
    info = plsc.get_sparse_core_info()
    nc, ns = info.num_cores, info.num_subcores
    nw = nc * ns
    bpw = _N_TOKENS // nw
    mesh = plsc.VectorSubcoreMesh(core_axis_name="c", subcore_axis_name="s")

    @functools.partial(
        pl.kernel,
        mesh=mesh,
        compiler_params=pltpu.CompilerParams(use_tc_tiling_on_sc=False),
        out_type=jax.ShapeDtypeStruct((_N_TOKENS, _DIM), jnp.float32),
        scratch_types=[
            pltpu.VMEM((bpw,), jnp.int32),
            pltpu.VMEM((bpw, _DIM), jnp.float32),
            pltpu.SemaphoreType.DMA,
        ],
    )
    def gather_kernel(table_hbm, idx_hbm, out_hbm, idx_v, rows_v, sem):
        wid = lax.axis_index("s") * nc + lax.axis_index("c")
        base = wid * bpw
        pltpu.sync_copy(idx_hbm.at[pl.ds(base, bpw)], idx_v)
        pltpu.async_copy(table_hbm.at[idx_v], rows_v, sem).wait()
        pltpu.sync_copy(rows_v, out_hbm.at[pl.ds(base, bpw)])

    return gather_kernel(table, idx)


def kernel(x, embeddings):
    table, idx = _tc_argmin(x, embeddings)
    return _sc_gather(table, idx)


# speedup vs baseline: 1.2482x; 1.0218x over previous
import functools

import jax
import jax.numpy as jnp
from jax import lax
from jax.experimental import pallas as pl
from jax.experimental.pallas import tpu as pltpu
from jax.experimental.pallas import tpu_sc as plsc

_N_TOKENS = 16384
_NUM_EMB = 8192
_DIM = 32
_TB = 2048


_LANES = 128
_TCHUNK = _NUM_EMB * _TB // _N_TOKENS
_PANEL = 1024


def _argmin_body(x_ref, e_ref, tbl_ref, idx_ref, em2_ref):
    @pl.when(pl.program_id(0) == 0)
    def _init():
        e = e_ref[...]
        em2_ref[0:_DIM] = e * -2.0
        e2 = jnp.sum(e * e, axis=0, keepdims=True)
        hi = e2.astype(jnp.bfloat16).astype(jnp.float32)
        r = e2 - hi
        mid = r.astype(jnp.bfloat16).astype(jnp.float32)
        lo = r - mid
        em2_ref[_DIM:_DIM + 1] = hi
        em2_ref[_DIM + 1:_DIM + 2] = mid
        em2_ref[_DIM + 2:_DIM + 3] = lo
        em2_ref[_DIM + 3:] = jnp.zeros((5, _NUM_EMB), jnp.float32)

    i = pl.program_id(0)
    tbl_ref[...] = e_ref[pl.ds(0, _DIM), pl.ds(i * _TCHUNK, _TCHUNK)].T

    ones3 = jnp.where(
        lax.broadcasted_iota(jnp.int32, (_TB, 8), 1) < 3,
        jnp.float32(1.0), jnp.float32(0.0),
    )
    xa = jnp.concatenate([x_ref[...], ones3], axis=1)

    run_min = None
    run_cid = jnp.zeros((_TB, _LANES), jnp.float32)
    for p in range(_NUM_EMB // _PANEL):
        dp = jnp.dot(
            xa, em2_ref[:, p * _PANEL:(p + 1) * _PANEL],
            preferred_element_type=jnp.float32,
        )
        for c in range(_PANEL // _LANES):
            dc = dp[:, c * _LANES:(c + 1) * _LANES]
            g = p * _PANEL + c * _LANES
            if run_min is None:
                run_min = dc
                continue
            pred = dc < run_min
            run_min = jnp.where(pred, dc, run_min)
            run_cid = jnp.where(pred, jnp.float32(g), run_cid)

    m = jnp.min(run_min, axis=1, keepdims=True)
    lane = lax.broadcasted_iota(
        jnp.int32, (_TB, _LANES), 1
    ).astype(jnp.float32)
    cand = jnp.where(
        run_min == m, run_cid + lane, jnp.float32(_NUM_EMB)
    )
    idxf = jnp.min(cand, axis=1)
    idx_ref[0] = idxf.astype(jnp.int32).reshape(_TB // _LANES, _LANES)


def _tc_argmin(x, embeddings):
    nb = _N_TOKENS // _TB
    out = pl.pallas_call(
        _argmin_body,
        grid=(nb,),
        in_specs=[
            pl.BlockSpec((_TB, _DIM), lambda i: (i, 0)),
            pl.BlockSpec((_DIM, _NUM_EMB), lambda i: (0, 0)),
        ],
        out_specs=[
            pl.BlockSpec((_TCHUNK, _DIM), lambda i: (i, 0)),
            pl.BlockSpec((1, _TB // _LANES, _LANES), lambda i: (i, 0, 0)),
        ],
        out_shape=[
            jax.ShapeDtypeStruct((_NUM_EMB, _DIM), jnp.float32),
            jax.ShapeDtypeStruct((nb, _TB // _LANES, _LANES), jnp.int32),
        ],
        scratch_shapes=[
            pltpu.VMEM((_DIM + 8, _NUM_EMB), jnp.float32),
        ],
    )(x, embeddings)
    return out[0], out[1].reshape(_N_TOKENS)


def _sc_gather(table, idx):
    info = plsc.get_sparse_core_info()
    nc, ns = info.num_cores, info.num_subcores
    nw = nc * ns
    bpw = _N_TOKENS // nw
    mesh = plsc.VectorSubcoreMesh(core_axis_name="c", subcore_axis_name="s")

    @functools.partial(
        pl.kernel,
        mesh=mesh,
        compiler_params=pltpu.CompilerParams(use_tc_tiling_on_sc=False),
        out_type=jax.ShapeDtypeStruct((_N_TOKENS, _DIM), jnp.float32),
        scratch_types=[
            pltpu.VMEM((bpw,), jnp.int32),
            pltpu.VMEM((bpw, _DIM), jnp.float32),
            pltpu.SemaphoreType.DMA,
        ],
    )
    def gather_kernel(table_hbm, idx_hbm, out_hbm, idx_v, rows_v, sem):
        wid = lax.axis_index("s") * nc + lax.axis_index("c")
        base = wid * bpw
        pltpu.sync_copy(idx_hbm.at[pl.ds(base, bpw)], idx_v)
        pltpu.async_copy(table_hbm.at[idx_v], rows_v, sem).wait()
        pltpu.sync_copy(rows_v, out_hbm.at[pl.ds(base, bpw)])

    return gather_kernel(table, idx)


def kernel(x, embeddings):
    table, idx = _tc_argmin(x, embeddings)
    return _sc_gather(table, idx)
